# Initial kernel scaffold; baseline (speedup 1.0000x reference)
import jax
import jax.numpy as jnp
from jax.experimental import pallas as pl
from jax.experimental.pallas import tpu as pltpu
from jax.experimental.pallas import tpu_sc as plsc

WINDOW = 128


def kernel(x, emb_weight):
    batch, hist = x.shape
    _, emb_dim = emb_weight.shape
    n_idx = batch * hist
    idx = x.reshape(1, n_idx).astype(jnp.int32)

    mesh = plsc.VectorSubcoreMesh(
        core_axis_name="core", subcore_axis_name="subcore"
    )

    @pl.kernel(
        out_type=jax.ShapeDtypeStruct((n_idx, emb_dim), jnp.float32),
        mesh=mesh,
    )
    def gather_kernel(table_hbm, i_hbm, o_hbm):
        def body(i_vmem, o_vmem):
            pltpu.sync_copy(table_hbm.at[i_vmem.at[0]], o_vmem)

        pltpu.emit_pipeline(
            body,
            grid=(n_idx // WINDOW,),
            in_specs=[
                pl.BlockSpec((1, WINDOW), index_map=lambda i: (0, i))
            ],
            out_specs=[
                pl.BlockSpec((WINDOW, emb_dim), index_map=lambda i: (i, 0))
            ],
            core_axis_name=("core", "subcore"),
            dimension_semantics=(pltpu.PARALLEL,),
        )(i_hbm, o_hbm)

    out = gather_kernel(emb_weight, idx)
    return out.reshape(batch, hist, emb_dim)


# speedup vs baseline: 1.7443x; 1.7443x over previous
import jax
import jax.numpy as jnp
from jax.experimental import pallas as pl
from jax.experimental.pallas import tpu as pltpu
from jax.experimental.pallas import tpu_sc as plsc

WINDOW = 128


def kernel(x, emb_weight):
    batch, hist = x.shape
    _, emb_dim = emb_weight.shape
    n_idx = batch * hist
    idx = x.reshape(1, n_idx).astype(jnp.int32)

    mesh = plsc.VectorSubcoreMesh(
        core_axis_name="core", subcore_axis_name="subcore"
    )

    @pl.kernel(
        out_type=jax.ShapeDtypeStruct((n_idx, emb_dim), jnp.float32),
        mesh=mesh,
        compiler_params=pltpu.CompilerParams(use_tc_tiling_on_sc=False),
    )
    def gather_kernel(table_hbm, i_hbm, o_hbm):
        def body(i_vmem, o_vmem):
            pltpu.sync_copy(table_hbm.at[i_vmem.at[0]], o_vmem)

        pltpu.emit_pipeline(
            body,
            grid=(n_idx // WINDOW,),
            in_specs=[
                pl.BlockSpec((1, WINDOW), index_map=lambda i: (0, i))
            ],
            out_specs=[
                pl.BlockSpec((WINDOW, emb_dim), index_map=lambda i: (i, 0))
            ],
            core_axis_name=("core", "subcore"),
            dimension_semantics=(pltpu.PARALLEL,),
        )(i_hbm, o_hbm)

    out = gather_kernel(emb_weight, idx)
    return out.reshape(batch, hist, emb_dim)


# speedup vs baseline: 1.8711x; 1.0727x over previous
import jax
import jax.numpy as jnp
from jax.experimental import pallas as pl
from jax.experimental.pallas import tpu as pltpu
from jax.experimental.pallas import tpu_sc as plsc

WINDOW = 512


def kernel(x, emb_weight):
    batch, hist = x.shape
    _, emb_dim = emb_weight.shape
    n_idx = batch * hist
    idx = x.reshape(1, n_idx).astype(jnp.int32)

    mesh = plsc.VectorSubcoreMesh(
        core_axis_name="core", subcore_axis_name="subcore"
    )

    @pl.kernel(
        out_type=jax.ShapeDtypeStruct((n_idx, emb_dim), jnp.float32),
        mesh=mesh,
        compiler_params=pltpu.CompilerParams(use_tc_tiling_on_sc=False),
    )
    def gather_kernel(table_hbm, i_hbm, o_hbm):
        def body(i_vmem, o_vmem):
            pltpu.sync_copy(table_hbm.at[i_vmem.at[0]], o_vmem)

        pltpu.emit_pipeline(
            body,
            grid=(n_idx // WINDOW,),
            in_specs=[
                pl.BlockSpec((1, WINDOW), index_map=lambda i: (0, i))
            ],
            out_specs=[
                pl.BlockSpec((WINDOW, emb_dim), index_map=lambda i: (i, 0))
            ],
            core_axis_name=("core", "subcore"),
            dimension_semantics=(pltpu.PARALLEL,),
        )(i_hbm, o_hbm)

    out = gather_kernel(emb_weight, idx)
    return out.reshape(batch, hist, emb_dim)
